# baseline (speedup 1.0000x reference)
import functools

import jax
import jax.numpy as jnp
from jax import lax
from jax.experimental import pallas as pl
from jax.experimental.pallas import tpu as pltpu
from jax.experimental.pallas import tpu_sc as plsc

_BLK = 128
_NEG = -1e30
_NUM_EXPERTS = 8


def _ln_router_body(x_ref, g_ref, b_ref, wp_ref,
                    nrm_ref, scl_ref, p_ref, m_ref, obuf, cumb, runn):
    f32 = jnp.float32
    i = pl.program_id(0)
    nblk = obuf.shape[0]
    r_i = lax.broadcasted_iota(jnp.int32, (128, 128), 0)
    c_i = lax.broadcasted_iota(jnp.int32, (128, 128), 1)
    ones_row = jnp.ones((1, 128), f32)

    @pl.when(i == 0)
    def _():
        runn[...] = jnp.zeros((1, 128), f32)

    @pl.when(i < nblk)
    def _():
        x = x_ref[...]
        mu = jnp.mean(x, axis=1, keepdims=True)
        xc = x - mu
        var = jnp.mean(xc * xc, axis=1, keepdims=True)
        nrm = xc * lax.rsqrt(var + 1e-5)
        nrm = nrm * g_ref[...] + b_ref[...]
        logits = jnp.dot(nrm, wp_ref[...], preferred_element_type=f32)
        logits = jnp.where(c_i < _NUM_EXPERTS, logits, _NEG)
        m = jnp.max(logits, axis=1, keepdims=True)
        eid_col = jnp.min(jnp.where(logits == m, c_i, 128), axis=1,
                          keepdims=True)
        nrm_ref[...] = nrm
        scl_ref[...] = nrm * jax.nn.sigmoid(m)
        onehot = (c_i == eid_col).astype(f32)
        cnt = jnp.dot(ones_row, onehot, preferred_element_type=f32)
        cumb[i] = runn[...]
        obuf[i] = onehot.astype(jnp.bfloat16)
        runn[...] = runn[...] + cnt

    @pl.when(i == nblk)
    def _():
        lstrict = (c_i < r_i).astype(f32)
        ustrict = (r_i < c_i).astype(f32)
        counts = runn[...]
        pc = jnp.floor((counts + (_BLK - 1.0)) / _BLK) * _BLK
        poff = jnp.dot(pc, ustrict, preferred_element_type=f32)

        shift_e = (r_i + _NUM_EXPERTS == c_i).astype(f32)
        lane = lax.broadcasted_iota(jnp.int32, (1, 128), 1)
        packed = jnp.where(lane < _NUM_EXPERTS, poff,
                           jnp.dot(pc / _BLK, shift_e, preferred_element_type=f32))
        m_ref[...] = packed.astype(jnp.int32).reshape(1, 1, 128)

        def pass1(b, carry):
            onehot = obuf[b].astype(f32)
            rank_before = jnp.dot(lstrict, onehot, preferred_element_type=f32)
            val = onehot * (rank_before + poff + cumb[b])
            prow = lax.dot_general(ones_row, val, (((1,), (1,)), ((), ())),
                                   preferred_element_type=f32)
            p_ref[b] = prow.astype(jnp.int32)
            return carry

        lax.fori_loop(0, nblk, pass1, 0)


def _ln_router(hs, gamma, beta, router_w):
    t, h = hs.shape
    nb = t // _BLK
    wp = jnp.zeros((h, 128), jnp.float32).at[:, : router_w.shape[1]].set(router_w)
    return pl.pallas_call(
        _ln_router_body,
        grid=(nb + 1,),
        in_specs=[
            pl.BlockSpec((_BLK, h), lambda b: (jnp.minimum(b, nb - 1), 0)),
            pl.BlockSpec((1, h), lambda b: (0, 0)),
            pl.BlockSpec((1, h), lambda b: (0, 0)),
            pl.BlockSpec((h, 128), lambda b: (0, 0)),
        ],
        out_specs=[
            pl.BlockSpec((_BLK, h), lambda b: (jnp.minimum(b, nb - 1), 0)),
            pl.BlockSpec((_BLK, h), lambda b: (jnp.minimum(b, nb - 1), 0)),
            pl.BlockSpec((nb, 1, 128), lambda b: (0, 0, 0)),
            pl.BlockSpec((1, 1, 128), lambda b: (0, 0, 0)),
        ],
        out_shape=[
            jax.ShapeDtypeStruct((t, h), jnp.float32),
            jax.ShapeDtypeStruct((t, h), jnp.float32),
            jax.ShapeDtypeStruct((nb, 1, 128), jnp.int32),
            jax.ShapeDtypeStruct((1, 1, 128), jnp.int32),
        ],
        scratch_shapes=[
            pltpu.VMEM((nb, 128, 128), jnp.bfloat16),
            pltpu.VMEM((nb, 1, 128), jnp.float32),
            pltpu.VMEM((1, 128), jnp.float32),
        ],
    )(hs, gamma.reshape(1, h), beta.reshape(1, h), wp)


def _make_sc_scatter(n_src, n_out, n_cols):
    info = plsc.get_sparse_core_info()
    nc, ns = info.num_cores, info.num_subcores
    nw = nc * ns
    bpw = n_src // nw
    mesh = plsc.VectorSubcoreMesh(core_axis_name="c", subcore_axis_name="s")

    @functools.partial(
        pl.kernel,
        mesh=mesh,
        out_type=jax.ShapeDtypeStruct((n_out, n_cols), jnp.float32),
        scratch_types=[
            pltpu.VMEM((bpw,), jnp.int32),
            pltpu.VMEM((bpw, n_cols), jnp.float32),
            pltpu.SemaphoreType.DMA,
        ],
    )
    def scatter_k(table_hbm, idx_hbm, out_hbm, idx_v, rows_v, sem):
        wid = lax.axis_index("s") * nc + lax.axis_index("c")
        base = wid * bpw
        pltpu.sync_copy(idx_hbm.at[pl.ds(base, bpw)], idx_v)
        pltpu.sync_copy(table_hbm.at[pl.ds(base, bpw)], rows_v)
        pltpu.async_copy(rows_v, out_hbm.at[idx_v], sem).wait()

    return scatter_k


def _make_sc_gather(n_out, n_cols):
    info = plsc.get_sparse_core_info()
    nc, ns = info.num_cores, info.num_subcores
    nw = nc * ns
    bpw = n_out // nw
    mesh = plsc.VectorSubcoreMesh(core_axis_name="c", subcore_axis_name="s")

    @functools.partial(
        pl.kernel,
        mesh=mesh,
        out_type=jax.ShapeDtypeStruct((n_out, n_cols), jnp.float32),
        scratch_types=[
            pltpu.VMEM((bpw,), jnp.int32),
            pltpu.VMEM((bpw, n_cols), jnp.float32),
            pltpu.SemaphoreType.DMA,
        ],
    )
    def gather_k(table_hbm, idx_hbm, out_hbm, idx_v, rows_v, sem):
        wid = lax.axis_index("s") * nc + lax.axis_index("c")
        base = wid * bpw
        pltpu.sync_copy(idx_hbm.at[pl.ds(base, bpw)], idx_v)
        pltpu.async_copy(table_hbm.at[idx_v], rows_v, sem).wait()
        pltpu.sync_copy(rows_v, out_hbm.at[pl.ds(base, bpw)])

    return gather_k


_FFB = 1024


def _expert_body(m_ref, x_ref, wg_ref, wu_ref, wd_ref, o_ref):
    e = pl.program_id(0)
    f = pl.program_id(1)
    poff = m_ref[e]
    nblk = m_ref[_NUM_EXPERTS + e]

    def body(j, carry):
        r0 = pl.multiple_of(poff + j * _BLK, _BLK)
        x = x_ref[pl.ds(r0, _BLK), :]
        g = jnp.dot(x, wg_ref[0], preferred_element_type=jnp.float32)
        u = jnp.dot(x, wu_ref[0], preferred_element_type=jnp.float32)
        part = jnp.dot(g * jax.nn.sigmoid(g) * u, wd_ref[0],
                       preferred_element_type=jnp.float32)

        @pl.when(f == 0)
        def _():
            o_ref[pl.ds(r0, _BLK), :] = part

        @pl.when(f != 0)
        def _():
            o_ref[pl.ds(r0, _BLK), :] = o_ref[pl.ds(r0, _BLK), :] + part

        return carry

    lax.fori_loop(0, nblk, body, 0)


def _grouped_ffn(meta128, xin, gate_up_proj, down_proj):
    npad, h = xin.shape
    ff = down_proj.shape[1]
    kf = ff // _FFB
    grid_spec = pltpu.PrefetchScalarGridSpec(
        num_scalar_prefetch=1,
        grid=(_NUM_EXPERTS, kf),
        in_specs=[
            pl.BlockSpec((npad, h), lambda e, f, m: (0, 0)),
            pl.BlockSpec((1, h, _FFB), lambda e, f, m: (e, 0, f)),
            pl.BlockSpec((1, h, _FFB), lambda e, f, m: (e, 0, f + kf)),
            pl.BlockSpec((1, _FFB, h), lambda e, f, m: (e, f, 0)),
        ],
        out_specs=pl.BlockSpec((npad, h), lambda e, f, m: (0, 0)),
    )
    return pl.pallas_call(
        _expert_body,
        grid_spec=grid_spec,
        out_shape=jax.ShapeDtypeStruct((npad, h), jnp.float32),
        compiler_params=pltpu.CompilerParams(
            vmem_limit_bytes=100 * 1024 * 1024,
        ),
    )(meta128, xin, gate_up_proj, gate_up_proj, down_proj)


def _shared_body0(hid_ref, nrm_ref, wg_ref, wu_ref, wd_ref, o_ref):
    x = nrm_ref[...]
    g = jnp.dot(x, wg_ref[...], preferred_element_type=jnp.float32)
    u = jnp.dot(x, wu_ref[...], preferred_element_type=jnp.float32)
    s = jnp.dot(g * jax.nn.sigmoid(g) * u, wd_ref[...],
                preferred_element_type=jnp.float32)
    o_ref[...] = hid_ref[...] + s


def _shared_body1(base_ref, rout_ref, nrm_ref, wg_ref, wu_ref, wd_ref, o_ref):
    x = nrm_ref[...]
    g = jnp.dot(x, wg_ref[...], preferred_element_type=jnp.float32)
    u = jnp.dot(x, wu_ref[...], preferred_element_type=jnp.float32)
    s = jnp.dot(g * jax.nn.sigmoid(g) * u, wd_ref[...],
                preferred_element_type=jnp.float32)
    o_ref[...] = base_ref[...] + rout_ref[...] + s


def _shared_ffn_half(residuals, nrm, wg, wu, wd, half):
    t, h = nrm.shape
    ff = wg.shape[1]
    fh = ff // 2
    nb = t // _BLK
    body = _shared_body0 if half == 0 else _shared_body1
    res_specs = [pl.BlockSpec((_BLK, h), lambda b: (b, 0)) for _ in residuals]
    return pl.pallas_call(
        body,
        grid=(nb,),
        in_specs=res_specs + [
            pl.BlockSpec((_BLK, h), lambda b: (b, 0)),
            pl.BlockSpec((h, fh), lambda b: (0, half)),
            pl.BlockSpec((h, fh), lambda b: (0, half)),
            pl.BlockSpec((fh, h), lambda b: (half, 0)),
        ],
        out_specs=pl.BlockSpec((_BLK, h), lambda b: (b, 0)),
        out_shape=jax.ShapeDtypeStruct((t, h), jnp.float32),
        compiler_params=pltpu.CompilerParams(
            vmem_limit_bytes=100 * 1024 * 1024,
        ),
    )(*residuals, nrm, wg, wu, wd)


def kernel(hidden_states, ln_gamma, ln_beta, router_w, gate_up_proj,
           down_proj, shared_gate, shared_up, shared_down):
    b, s, h = hidden_states.shape
    t = b * s
    e = router_w.shape[1]
    nb = t // _BLK + e
    npad = nb * _BLK
    hs = hidden_states.reshape(t, h)

    nrm, scl, p3, meta3 = _ln_router(hs, ln_gamma, ln_beta, router_w)
    p = p3.reshape(t)
    meta128 = meta3.reshape(-1)

    xin_sorted = _make_sc_scatter(t, npad, h)(scl, p)
    routed_sorted = _grouped_ffn(meta128, xin_sorted, gate_up_proj, down_proj)
    routed = _make_sc_gather(t, h)(routed_sorted, p)

    base = _shared_ffn_half([hs], nrm, shared_gate, shared_up, shared_down, 0)
    out = _shared_ffn_half([base, routed], nrm,
                           shared_gate, shared_up, shared_down, 1)
    return out.reshape(b, s, h)


# speedup vs baseline: 1.0083x; 1.0083x over previous
import functools

import jax
import jax.numpy as jnp
from jax import lax
from jax.experimental import pallas as pl
from jax.experimental.pallas import tpu as pltpu
from jax.experimental.pallas import tpu_sc as plsc

_BLK = 128
_NEG = -1e30
_NUM_EXPERTS = 8


def _ln_router_body(x_ref, g_ref, b_ref, wp_ref,
                    nrm_ref, scl_ref, p_ref, m_ref, obuf, cumb, runn):
    f32 = jnp.float32
    i = pl.program_id(0)
    nblk = obuf.shape[0]
    r_i = lax.broadcasted_iota(jnp.int32, (128, 128), 0)
    c_i = lax.broadcasted_iota(jnp.int32, (128, 128), 1)
    ones_row = jnp.ones((1, 128), f32)

    @pl.when(i == 0)
    def _():
        runn[...] = jnp.zeros((1, 128), f32)

    @pl.when(i < nblk)
    def _():
        x = x_ref[...]
        mu = jnp.mean(x, axis=1, keepdims=True)
        xc = x - mu
        var = jnp.mean(xc * xc, axis=1, keepdims=True)
        nrm = xc * lax.rsqrt(var + 1e-5)
        nrm = nrm * g_ref[...] + b_ref[...]
        logits = jnp.dot(nrm, wp_ref[...], preferred_element_type=f32)
        logits = jnp.where(c_i < _NUM_EXPERTS, logits, _NEG)
        m = jnp.max(logits, axis=1, keepdims=True)
        eid_col = jnp.min(jnp.where(logits == m, c_i, 128), axis=1,
                          keepdims=True)
        nrm_ref[...] = nrm
        scl_ref[...] = nrm * jax.nn.sigmoid(m)
        onehot = (c_i == eid_col).astype(f32)
        cnt = jnp.dot(ones_row, onehot, preferred_element_type=f32)
        cumb[i] = runn[...]
        obuf[i] = onehot.astype(jnp.bfloat16)
        runn[...] = runn[...] + cnt

    @pl.when(i == nblk)
    def _():
        lstrict = (c_i < r_i).astype(f32)
        ustrict = (r_i < c_i).astype(f32)
        counts = runn[...]
        pc = jnp.floor((counts + (_BLK - 1.0)) / _BLK) * _BLK
        poff = jnp.dot(pc, ustrict, preferred_element_type=f32)

        shift_e = (r_i + _NUM_EXPERTS == c_i).astype(f32)
        lane = lax.broadcasted_iota(jnp.int32, (1, 128), 1)
        packed = jnp.where(lane < _NUM_EXPERTS, poff,
                           jnp.dot(pc / _BLK, shift_e, preferred_element_type=f32))
        m_ref[...] = packed.astype(jnp.int32).reshape(1, 1, 128)

        def pass1(b, carry):
            onehot = obuf[b].astype(f32)
            rank_before = jnp.dot(lstrict, onehot, preferred_element_type=f32)
            val = rank_before + poff + cumb[b]
            pcol = jnp.sum(onehot * val, axis=1)
            p_ref[b] = pcol.reshape(1, 128).astype(jnp.int32)
            return carry

        lax.fori_loop(0, nblk, pass1, 0)


def _ln_router(hs, gamma, beta, router_w):
    t, h = hs.shape
    nb = t // _BLK
    wp = jnp.zeros((h, 128), jnp.float32).at[:, : router_w.shape[1]].set(router_w)
    return pl.pallas_call(
        _ln_router_body,
        grid=(nb + 1,),
        in_specs=[
            pl.BlockSpec((_BLK, h), lambda b: (jnp.minimum(b, nb - 1), 0)),
            pl.BlockSpec((1, h), lambda b: (0, 0)),
            pl.BlockSpec((1, h), lambda b: (0, 0)),
            pl.BlockSpec((h, 128), lambda b: (0, 0)),
        ],
        out_specs=[
            pl.BlockSpec((_BLK, h), lambda b: (jnp.minimum(b, nb - 1), 0)),
            pl.BlockSpec((_BLK, h), lambda b: (jnp.minimum(b, nb - 1), 0)),
            pl.BlockSpec((nb, 1, 128), lambda b: (0, 0, 0)),
            pl.BlockSpec((1, 1, 128), lambda b: (0, 0, 0)),
        ],
        out_shape=[
            jax.ShapeDtypeStruct((t, h), jnp.float32),
            jax.ShapeDtypeStruct((t, h), jnp.float32),
            jax.ShapeDtypeStruct((nb, 1, 128), jnp.int32),
            jax.ShapeDtypeStruct((1, 1, 128), jnp.int32),
        ],
        scratch_shapes=[
            pltpu.VMEM((nb, 128, 128), jnp.bfloat16),
            pltpu.VMEM((nb, 1, 128), jnp.float32),
            pltpu.VMEM((1, 128), jnp.float32),
        ],
    )(hs, gamma.reshape(1, h), beta.reshape(1, h), wp)


def _make_sc_scatter(n_src, n_out, n_cols):
    info = plsc.get_sparse_core_info()
    nc, ns = info.num_cores, info.num_subcores
    nw = nc * ns
    bpw = n_src // nw
    mesh = plsc.VectorSubcoreMesh(core_axis_name="c", subcore_axis_name="s")

    @functools.partial(
        pl.kernel,
        mesh=mesh,
        out_type=jax.ShapeDtypeStruct((n_out, n_cols), jnp.float32),
        scratch_types=[
            pltpu.VMEM((bpw,), jnp.int32),
            pltpu.VMEM((bpw, n_cols), jnp.float32),
            pltpu.SemaphoreType.DMA,
        ],
    )
    def scatter_k(table_hbm, idx_hbm, out_hbm, idx_v, rows_v, sem):
        wid = lax.axis_index("s") * nc + lax.axis_index("c")
        base = wid * bpw
        pltpu.sync_copy(idx_hbm.at[pl.ds(base, bpw)], idx_v)
        pltpu.sync_copy(table_hbm.at[pl.ds(base, bpw)], rows_v)
        pltpu.async_copy(rows_v, out_hbm.at[idx_v], sem).wait()

    return scatter_k


def _make_sc_gather(n_out, n_cols):
    info = plsc.get_sparse_core_info()
    nc, ns = info.num_cores, info.num_subcores
    nw = nc * ns
    bpw = n_out // nw
    mesh = plsc.VectorSubcoreMesh(core_axis_name="c", subcore_axis_name="s")

    @functools.partial(
        pl.kernel,
        mesh=mesh,
        out_type=jax.ShapeDtypeStruct((n_out, n_cols), jnp.float32),
        scratch_types=[
            pltpu.VMEM((bpw,), jnp.int32),
            pltpu.VMEM((bpw, n_cols), jnp.float32),
            pltpu.SemaphoreType.DMA,
        ],
    )
    def gather_k(table_hbm, idx_hbm, out_hbm, idx_v, rows_v, sem):
        wid = lax.axis_index("s") * nc + lax.axis_index("c")
        base = wid * bpw
        pltpu.sync_copy(idx_hbm.at[pl.ds(base, bpw)], idx_v)
        pltpu.async_copy(table_hbm.at[idx_v], rows_v, sem).wait()
        pltpu.sync_copy(rows_v, out_hbm.at[pl.ds(base, bpw)])

    return gather_k


_FFB = 1024


def _expert_body(m_ref, x_ref, wg_ref, wu_ref, wd_ref, o_ref):
    e = pl.program_id(0)
    f = pl.program_id(1)
    poff = m_ref[e]
    nblk = m_ref[_NUM_EXPERTS + e]

    def body(j, carry):
        r0 = pl.multiple_of(poff + j * _BLK, _BLK)
        x = x_ref[pl.ds(r0, _BLK), :]
        g = jnp.dot(x, wg_ref[0], preferred_element_type=jnp.float32)
        u = jnp.dot(x, wu_ref[0], preferred_element_type=jnp.float32)
        part = jnp.dot(g * jax.nn.sigmoid(g) * u, wd_ref[0],
                       preferred_element_type=jnp.float32)

        @pl.when(f == 0)
        def _():
            o_ref[pl.ds(r0, _BLK), :] = part

        @pl.when(f != 0)
        def _():
            o_ref[pl.ds(r0, _BLK), :] = o_ref[pl.ds(r0, _BLK), :] + part

        return carry

    lax.fori_loop(0, nblk, body, 0)


def _grouped_ffn(meta128, xin, gate_up_proj, down_proj):
    npad, h = xin.shape
    ff = down_proj.shape[1]
    kf = ff // _FFB
    grid_spec = pltpu.PrefetchScalarGridSpec(
        num_scalar_prefetch=1,
        grid=(_NUM_EXPERTS, kf),
        in_specs=[
            pl.BlockSpec((npad, h), lambda e, f, m: (0, 0)),
            pl.BlockSpec((1, h, _FFB), lambda e, f, m: (e, 0, f)),
            pl.BlockSpec((1, h, _FFB), lambda e, f, m: (e, 0, f + kf)),
            pl.BlockSpec((1, _FFB, h), lambda e, f, m: (e, f, 0)),
        ],
        out_specs=pl.BlockSpec((npad, h), lambda e, f, m: (0, 0)),
    )
    return pl.pallas_call(
        _expert_body,
        grid_spec=grid_spec,
        out_shape=jax.ShapeDtypeStruct((npad, h), jnp.float32),
        compiler_params=pltpu.CompilerParams(
            vmem_limit_bytes=100 * 1024 * 1024,
        ),
    )(meta128, xin, gate_up_proj, gate_up_proj, down_proj)


def _shared_body0(hid_ref, nrm_ref, wg_ref, wu_ref, wd_ref, o_ref):
    x = nrm_ref[...]
    g = jnp.dot(x, wg_ref[...], preferred_element_type=jnp.float32)
    u = jnp.dot(x, wu_ref[...], preferred_element_type=jnp.float32)
    s = jnp.dot(g * jax.nn.sigmoid(g) * u, wd_ref[...],
                preferred_element_type=jnp.float32)
    o_ref[...] = hid_ref[...] + s


def _shared_body1(base_ref, rout_ref, nrm_ref, wg_ref, wu_ref, wd_ref, o_ref):
    x = nrm_ref[...]
    g = jnp.dot(x, wg_ref[...], preferred_element_type=jnp.float32)
    u = jnp.dot(x, wu_ref[...], preferred_element_type=jnp.float32)
    s = jnp.dot(g * jax.nn.sigmoid(g) * u, wd_ref[...],
                preferred_element_type=jnp.float32)
    o_ref[...] = base_ref[...] + rout_ref[...] + s


def _shared_ffn_half(residuals, nrm, wg, wu, wd, half):
    t, h = nrm.shape
    ff = wg.shape[1]
    fh = ff // 2
    nb = t // _BLK
    body = _shared_body0 if half == 0 else _shared_body1
    res_specs = [pl.BlockSpec((_BLK, h), lambda b: (b, 0)) for _ in residuals]
    return pl.pallas_call(
        body,
        grid=(nb,),
        in_specs=res_specs + [
            pl.BlockSpec((_BLK, h), lambda b: (b, 0)),
            pl.BlockSpec((h, fh), lambda b: (0, half)),
            pl.BlockSpec((h, fh), lambda b: (0, half)),
            pl.BlockSpec((fh, h), lambda b: (half, 0)),
        ],
        out_specs=pl.BlockSpec((_BLK, h), lambda b: (b, 0)),
        out_shape=jax.ShapeDtypeStruct((t, h), jnp.float32),
        compiler_params=pltpu.CompilerParams(
            vmem_limit_bytes=100 * 1024 * 1024,
        ),
    )(*residuals, nrm, wg, wu, wd)


def kernel(hidden_states, ln_gamma, ln_beta, router_w, gate_up_proj,
           down_proj, shared_gate, shared_up, shared_down):
    b, s, h = hidden_states.shape
    t = b * s
    e = router_w.shape[1]
    nb = t // _BLK + e
    npad = nb * _BLK
    hs = hidden_states.reshape(t, h)

    nrm, scl, p3, meta3 = _ln_router(hs, ln_gamma, ln_beta, router_w)
    p = p3.reshape(t)
    meta128 = meta3.reshape(-1)

    xin_sorted = _make_sc_scatter(t, npad, h)(scl, p)
    routed_sorted = _grouped_ffn(meta128, xin_sorted, gate_up_proj, down_proj)
    routed = _make_sc_gather(t, h)(routed_sorted, p)

    base = _shared_ffn_half([hs], nrm, shared_gate, shared_up, shared_down, 0)
    out = _shared_ffn_half([base, routed], nrm,
                           shared_gate, shared_up, shared_down, 1)
    return out.reshape(b, s, h)


# speedup vs baseline: 1.0653x; 1.0565x over previous
import functools

import jax
import jax.numpy as jnp
from jax import lax
from jax.experimental import pallas as pl
from jax.experimental.pallas import tpu as pltpu
from jax.experimental.pallas import tpu_sc as plsc

_BLK = 128
_NEG = -1e30
_NUM_EXPERTS = 8


def _ln_router_body(x_ref, g_ref, b_ref, wp_ref,
                    nrm_ref, scl_ref, p_ref, m_ref, obuf, cumb, runn):
    f32 = jnp.float32
    i = pl.program_id(0)
    nblk = obuf.shape[0]
    r_i = lax.broadcasted_iota(jnp.int32, (128, 128), 0)
    c_i = lax.broadcasted_iota(jnp.int32, (128, 128), 1)
    ones_row = jnp.ones((1, 128), f32)

    @pl.when(i == 0)
    def _():
        runn[...] = jnp.zeros((1, 128), f32)

    @pl.when(i < nblk)
    def _():
        x = x_ref[...]
        mu = jnp.mean(x, axis=1, keepdims=True)
        xc = x - mu
        var = jnp.mean(xc * xc, axis=1, keepdims=True)
        nrm = xc * lax.rsqrt(var + 1e-5)
        nrm = nrm * g_ref[...] + b_ref[...]
        logits = jnp.dot(nrm, wp_ref[...], preferred_element_type=f32)
        logits = jnp.where(c_i < _NUM_EXPERTS, logits, _NEG)
        m = jnp.max(logits, axis=1, keepdims=True)
        eid_col = jnp.min(jnp.where(logits == m, c_i, 128), axis=1,
                          keepdims=True)
        nrm_ref[...] = nrm
        scl_ref[...] = nrm * jax.nn.sigmoid(m)
        onehot = (c_i == eid_col).astype(f32)
        cnt = jnp.dot(ones_row, onehot, preferred_element_type=f32)
        cumb[i] = runn[...]
        obuf[i] = onehot.astype(jnp.bfloat16)
        runn[...] = runn[...] + cnt

    @pl.when(i == nblk)
    def _():
        lstrict = (c_i < r_i).astype(f32)
        ustrict = (r_i < c_i).astype(f32)
        counts = runn[...]
        pc = jnp.floor((counts + (_BLK - 1.0)) / _BLK) * _BLK
        poff = jnp.dot(pc, ustrict, preferred_element_type=f32)

        shift_e = (r_i + _NUM_EXPERTS == c_i).astype(f32)
        lane = lax.broadcasted_iota(jnp.int32, (1, 128), 1)
        packed = jnp.where(lane < _NUM_EXPERTS, poff,
                           jnp.dot(pc / _BLK, shift_e, preferred_element_type=f32))
        m_ref[...] = packed.astype(jnp.int32).reshape(1, 1, 128)

        def pass1(b, carry):
            onehot = obuf[b].astype(f32)
            rank_before = jnp.dot(lstrict, onehot, preferred_element_type=f32)
            val = rank_before + poff + cumb[b]
            pcol = jnp.sum(onehot * val, axis=1)
            p_ref[b] = pcol.reshape(1, 128).astype(jnp.int32)
            return carry

        lax.fori_loop(0, nblk, pass1, 0)


def _ln_router(hs, gamma, beta, router_w):
    t, h = hs.shape
    nb = t // _BLK
    wp = jnp.zeros((h, 128), jnp.float32).at[:, : router_w.shape[1]].set(router_w)
    return pl.pallas_call(
        _ln_router_body,
        grid=(nb + 1,),
        in_specs=[
            pl.BlockSpec((_BLK, h), lambda b: (jnp.minimum(b, nb - 1), 0)),
            pl.BlockSpec((1, h), lambda b: (0, 0)),
            pl.BlockSpec((1, h), lambda b: (0, 0)),
            pl.BlockSpec((h, 128), lambda b: (0, 0)),
        ],
        out_specs=[
            pl.BlockSpec((_BLK, h), lambda b: (jnp.minimum(b, nb - 1), 0)),
            pl.BlockSpec((_BLK, h), lambda b: (jnp.minimum(b, nb - 1), 0)),
            pl.BlockSpec((nb, 1, 128), lambda b: (0, 0, 0)),
            pl.BlockSpec((1, 1, 128), lambda b: (0, 0, 0)),
        ],
        out_shape=[
            jax.ShapeDtypeStruct((t, h), jnp.float32),
            jax.ShapeDtypeStruct((t, h), jnp.float32),
            jax.ShapeDtypeStruct((nb, 1, 128), jnp.int32),
            jax.ShapeDtypeStruct((1, 1, 128), jnp.int32),
        ],
        scratch_shapes=[
            pltpu.VMEM((nb, 128, 128), jnp.bfloat16),
            pltpu.VMEM((nb, 1, 128), jnp.float32),
            pltpu.VMEM((1, 128), jnp.float32),
        ],
    )(hs, gamma.reshape(1, h), beta.reshape(1, h), wp)


def _make_sc_scatter(n_src, n_out, n_cols):
    info = plsc.get_sparse_core_info()
    nc, ns = info.num_cores, info.num_subcores
    nw = nc * ns
    bpw = n_src // nw
    mesh = plsc.VectorSubcoreMesh(core_axis_name="c", subcore_axis_name="s")

    @functools.partial(
        pl.kernel,
        mesh=mesh,
        out_type=jax.ShapeDtypeStruct((n_out, n_cols), jnp.float32),
        scratch_types=[
            pltpu.VMEM((bpw,), jnp.int32),
            pltpu.VMEM((bpw, n_cols), jnp.float32),
            pltpu.SemaphoreType.DMA,
        ],
    )
    def scatter_k(table_hbm, idx_hbm, out_hbm, idx_v, rows_v, sem):
        wid = lax.axis_index("s") * nc + lax.axis_index("c")
        base = wid * bpw
        pltpu.sync_copy(idx_hbm.at[pl.ds(base, bpw)], idx_v)
        pltpu.sync_copy(table_hbm.at[pl.ds(base, bpw)], rows_v)
        pltpu.async_copy(rows_v, out_hbm.at[idx_v], sem).wait()

    return scatter_k


def _make_sc_gather(n_out, n_cols):
    info = plsc.get_sparse_core_info()
    nc, ns = info.num_cores, info.num_subcores
    nw = nc * ns
    bpw = n_out // nw
    mesh = plsc.VectorSubcoreMesh(core_axis_name="c", subcore_axis_name="s")

    @functools.partial(
        pl.kernel,
        mesh=mesh,
        out_type=jax.ShapeDtypeStruct((n_out, n_cols), jnp.float32),
        scratch_types=[
            pltpu.VMEM((bpw,), jnp.int32),
            pltpu.VMEM((bpw, n_cols), jnp.float32),
            pltpu.SemaphoreType.DMA,
        ],
    )
    def gather_k(table_hbm, idx_hbm, out_hbm, idx_v, rows_v, sem):
        wid = lax.axis_index("s") * nc + lax.axis_index("c")
        base = wid * bpw
        pltpu.sync_copy(idx_hbm.at[pl.ds(base, bpw)], idx_v)
        pltpu.async_copy(table_hbm.at[idx_v], rows_v, sem).wait()
        pltpu.sync_copy(rows_v, out_hbm.at[pl.ds(base, bpw)])

    return gather_k


_FFB = 1024


def _expert_body(m_ref, x_ref, wg_ref, wu_ref, wd_ref, o_ref):
    e = pl.program_id(0)
    f = pl.program_id(1)
    poff = m_ref[e]
    nblk = m_ref[_NUM_EXPERTS + e]

    def body(j, carry):
        r0 = pl.multiple_of(poff + j * _BLK, _BLK)
        x = x_ref[pl.ds(r0, _BLK), :]
        g = jnp.dot(x, wg_ref[0], preferred_element_type=jnp.float32)
        u = jnp.dot(x, wu_ref[0], preferred_element_type=jnp.float32)
        part = jnp.dot(g * jax.nn.sigmoid(g) * u, wd_ref[0],
                       preferred_element_type=jnp.float32)

        @pl.when(f == 0)
        def _():
            o_ref[pl.ds(r0, _BLK), :] = part

        @pl.when(f != 0)
        def _():
            o_ref[pl.ds(r0, _BLK), :] = o_ref[pl.ds(r0, _BLK), :] + part

        return carry

    lax.fori_loop(0, nblk, body, 0)


def _grouped_ffn(meta128, xin, gate_up_proj, down_proj):
    npad, h = xin.shape
    ff = down_proj.shape[1]
    kf = ff // _FFB
    grid_spec = pltpu.PrefetchScalarGridSpec(
        num_scalar_prefetch=1,
        grid=(_NUM_EXPERTS, kf),
        in_specs=[
            pl.BlockSpec((npad, h), lambda e, f, m: (0, 0)),
            pl.BlockSpec((1, h, _FFB), lambda e, f, m: (e, 0, f)),
            pl.BlockSpec((1, h, _FFB), lambda e, f, m: (e, 0, f + kf)),
            pl.BlockSpec((1, _FFB, h), lambda e, f, m: (e, f, 0)),
        ],
        out_specs=pl.BlockSpec((npad, h), lambda e, f, m: (0, 0)),
    )
    return pl.pallas_call(
        _expert_body,
        grid_spec=grid_spec,
        out_shape=jax.ShapeDtypeStruct((npad, h), jnp.float32),
        compiler_params=pltpu.CompilerParams(
            vmem_limit_bytes=100 * 1024 * 1024,
        ),
    )(meta128, xin, gate_up_proj, gate_up_proj, down_proj)


def _shared_body(hid_ref, rout_ref, nrm_ref, wg_ref, wu_ref, wd_ref, o_ref):
    x = nrm_ref[...]
    g = jnp.dot(x, wg_ref[...], preferred_element_type=jnp.float32)
    u = jnp.dot(x, wu_ref[...], preferred_element_type=jnp.float32)
    s = jnp.dot(g * jax.nn.sigmoid(g) * u, wd_ref[...],
                preferred_element_type=jnp.float32)
    o_ref[...] = hid_ref[...] + rout_ref[...] + s


def _shared_ffn(hid, routed, nrm, wg, wu, wd):
    t, h = nrm.shape
    ff = wg.shape[1]
    nb = t // _BLK
    return pl.pallas_call(
        _shared_body,
        grid=(nb,),
        in_specs=[
            pl.BlockSpec((_BLK, h), lambda b: (b, 0)),
            pl.BlockSpec((_BLK, h), lambda b: (b, 0)),
            pl.BlockSpec((_BLK, h), lambda b: (b, 0)),
            pl.BlockSpec((h, ff), lambda b: (0, 0)),
            pl.BlockSpec((h, ff), lambda b: (0, 0)),
            pl.BlockSpec((ff, h), lambda b: (0, 0)),
        ],
        out_specs=pl.BlockSpec((_BLK, h), lambda b: (b, 0)),
        out_shape=jax.ShapeDtypeStruct((t, h), jnp.float32),
        compiler_params=pltpu.CompilerParams(
            vmem_limit_bytes=100 * 1024 * 1024,
        ),
    )(hid, routed, nrm, wg, wu, wd)


def kernel(hidden_states, ln_gamma, ln_beta, router_w, gate_up_proj,
           down_proj, shared_gate, shared_up, shared_down):
    b, s, h = hidden_states.shape
    t = b * s
    e = router_w.shape[1]
    nb = t // _BLK + e
    npad = nb * _BLK
    hs = hidden_states.reshape(t, h)

    nrm, scl, p3, meta3 = _ln_router(hs, ln_gamma, ln_beta, router_w)
    p = p3.reshape(t)
    meta128 = meta3.reshape(-1)

    xin_sorted = _make_sc_scatter(t, npad, h)(scl, p)
    routed_sorted = _grouped_ffn(meta128, xin_sorted, gate_up_proj, down_proj)
    routed = _make_sc_gather(t, h)(routed_sorted, p)

    out = _shared_ffn(hs, routed, nrm, shared_gate, shared_up, shared_down)
    return out.reshape(b, s, h)


# speedup vs baseline: 1.1099x; 1.0418x over previous
import functools

import jax
import jax.numpy as jnp
from jax import lax
from jax.experimental import pallas as pl
from jax.experimental.pallas import tpu as pltpu
from jax.experimental.pallas import tpu_sc as plsc

_BLK = 128
_NEG = -1e30
_NUM_EXPERTS = 8


_LNB = 256


def _ln_router_body(x_ref, g_ref, b_ref, wp_ref,
                    nrm_ref, scl_ref, p_ref, m_ref, obuf, cumb, runn):
    f32 = jnp.float32
    i = pl.program_id(0)
    nblk = obuf.shape[0]
    c_i = lax.broadcasted_iota(jnp.int32, (_LNB, 128), 1)
    ones_row = jnp.ones((1, _LNB), f32)

    @pl.when(i == 0)
    def _():
        runn[...] = jnp.zeros((1, 128), f32)

    @pl.when(i < nblk)
    def _():
        x = x_ref[...]
        mu = jnp.mean(x, axis=1, keepdims=True)
        xc = x - mu
        var = jnp.mean(xc * xc, axis=1, keepdims=True)
        nrm = xc * lax.rsqrt(var + 1e-5)
        nrm = nrm * g_ref[...] + b_ref[...]
        logits = jnp.dot(nrm, wp_ref[...], preferred_element_type=f32)
        logits = jnp.where(c_i < _NUM_EXPERTS, logits, _NEG)
        m = jnp.max(logits, axis=1, keepdims=True)
        eid_col = jnp.min(jnp.where(logits == m, c_i, 128), axis=1,
                          keepdims=True)
        nrm_ref[...] = nrm
        scl_ref[...] = nrm * jax.nn.sigmoid(m)
        onehot = (c_i == eid_col).astype(f32)
        cnt = jnp.dot(ones_row, onehot, preferred_element_type=f32)
        cumb[i] = runn[...]
        obuf[i] = onehot.astype(jnp.bfloat16)
        runn[...] = runn[...] + cnt

    @pl.when(i == nblk)
    def _():
        rt = lax.broadcasted_iota(jnp.int32, (_LNB, _LNB), 0)
        ct = lax.broadcasted_iota(jnp.int32, (_LNB, _LNB), 1)
        lstrict = (ct < rt).astype(f32)
        r_e = lax.broadcasted_iota(jnp.int32, (128, 128), 0)
        c_e = lax.broadcasted_iota(jnp.int32, (128, 128), 1)
        ustrict = (r_e < c_e).astype(f32)
        counts = runn[...]
        pc = jnp.floor((counts + (_BLK - 1.0)) / _BLK) * _BLK
        poff = jnp.dot(pc, ustrict, preferred_element_type=f32)

        shift_e = (r_e + _NUM_EXPERTS == c_e).astype(f32)
        lane = lax.broadcasted_iota(jnp.int32, (1, 128), 1)
        packed = jnp.where(lane < _NUM_EXPERTS, poff,
                           jnp.dot(pc / _BLK, shift_e, preferred_element_type=f32))
        m_ref[...] = packed.astype(jnp.int32).reshape(1, 1, 128)

        def pass1(b, carry):
            onehot = obuf[b].astype(f32)
            rank_before = jnp.dot(lstrict, onehot, preferred_element_type=f32)
            val = rank_before + poff + cumb[b]
            pcol = jnp.sum(onehot * val, axis=1)
            p_ref[b] = pcol.reshape(_LNB // 128, 128).astype(jnp.int32)
            return carry

        lax.fori_loop(0, nblk, pass1, 0)


def _ln_router(hs, gamma, beta, router_w):
    t, h = hs.shape
    nb = t // _LNB
    rows = _LNB // 128
    wp = jnp.zeros((h, 128), jnp.float32).at[:, : router_w.shape[1]].set(router_w)
    return pl.pallas_call(
        _ln_router_body,
        grid=(nb + 1,),
        in_specs=[
            pl.BlockSpec((_LNB, h), lambda b: (jnp.minimum(b, nb - 1), 0)),
            pl.BlockSpec((1, h), lambda b: (0, 0)),
            pl.BlockSpec((1, h), lambda b: (0, 0)),
            pl.BlockSpec((h, 128), lambda b: (0, 0)),
        ],
        out_specs=[
            pl.BlockSpec((_LNB, h), lambda b: (jnp.minimum(b, nb - 1), 0)),
            pl.BlockSpec((_LNB, h), lambda b: (jnp.minimum(b, nb - 1), 0)),
            pl.BlockSpec((nb, rows, 128), lambda b: (0, 0, 0)),
            pl.BlockSpec((1, 1, 128), lambda b: (0, 0, 0)),
        ],
        out_shape=[
            jax.ShapeDtypeStruct((t, h), jnp.float32),
            jax.ShapeDtypeStruct((t, h), jnp.float32),
            jax.ShapeDtypeStruct((nb, rows, 128), jnp.int32),
            jax.ShapeDtypeStruct((1, 1, 128), jnp.int32),
        ],
        scratch_shapes=[
            pltpu.VMEM((nb, _LNB, 128), jnp.bfloat16),
            pltpu.VMEM((nb, 1, 128), jnp.float32),
            pltpu.VMEM((1, 128), jnp.float32),
        ],
    )(hs, gamma.reshape(1, h), beta.reshape(1, h), wp)


def _make_sc_scatter(n_src, n_out, n_cols):
    info = plsc.get_sparse_core_info()
    nc, ns = info.num_cores, info.num_subcores
    nw = nc * ns
    bpw = n_src // nw
    mesh = plsc.VectorSubcoreMesh(core_axis_name="c", subcore_axis_name="s")

    @functools.partial(
        pl.kernel,
        mesh=mesh,
        out_type=jax.ShapeDtypeStruct((n_out, n_cols), jnp.float32),
        scratch_types=[
            pltpu.VMEM((bpw,), jnp.int32),
            pltpu.VMEM((bpw, n_cols), jnp.float32),
            pltpu.SemaphoreType.DMA,
        ],
    )
    def scatter_k(table_hbm, idx_hbm, out_hbm, idx_v, rows_v, sem):
        wid = lax.axis_index("s") * nc + lax.axis_index("c")
        base = wid * bpw
        pltpu.sync_copy(idx_hbm.at[pl.ds(base, bpw)], idx_v)
        pltpu.sync_copy(table_hbm.at[pl.ds(base, bpw)], rows_v)
        pltpu.async_copy(rows_v, out_hbm.at[idx_v], sem).wait()

    return scatter_k


def _make_sc_gather(n_out, n_cols):
    info = plsc.get_sparse_core_info()
    nc, ns = info.num_cores, info.num_subcores
    nw = nc * ns
    bpw = n_out // nw
    mesh = plsc.VectorSubcoreMesh(core_axis_name="c", subcore_axis_name="s")

    @functools.partial(
        pl.kernel,
        mesh=mesh,
        out_type=jax.ShapeDtypeStruct((n_out, n_cols), jnp.float32),
        scratch_types=[
            pltpu.VMEM((bpw,), jnp.int32),
            pltpu.VMEM((bpw, n_cols), jnp.float32),
            pltpu.SemaphoreType.DMA,
        ],
    )
    def gather_k(table_hbm, idx_hbm, out_hbm, idx_v, rows_v, sem):
        wid = lax.axis_index("s") * nc + lax.axis_index("c")
        base = wid * bpw
        pltpu.sync_copy(idx_hbm.at[pl.ds(base, bpw)], idx_v)
        pltpu.async_copy(table_hbm.at[idx_v], rows_v, sem).wait()
        pltpu.sync_copy(rows_v, out_hbm.at[pl.ds(base, bpw)])

    return gather_k


_FFB = 1024


def _expert_body(m_ref, x_ref, wg_ref, wu_ref, wd_ref, o_ref):
    e = pl.program_id(0)
    f = pl.program_id(1)
    poff = m_ref[e]
    nblk = m_ref[_NUM_EXPERTS + e]

    def body(j, carry):
        r0 = pl.multiple_of(poff + j * _BLK, _BLK)
        x = x_ref[pl.ds(r0, _BLK), :]
        g = jnp.dot(x, wg_ref[0], preferred_element_type=jnp.float32)
        u = jnp.dot(x, wu_ref[0], preferred_element_type=jnp.float32)
        part = jnp.dot(g * jax.nn.sigmoid(g) * u, wd_ref[0],
                       preferred_element_type=jnp.float32)

        @pl.when(f == 0)
        def _():
            o_ref[pl.ds(r0, _BLK), :] = part

        @pl.when(f != 0)
        def _():
            o_ref[pl.ds(r0, _BLK), :] = o_ref[pl.ds(r0, _BLK), :] + part

        return carry

    lax.fori_loop(0, nblk, body, 0)


def _grouped_ffn(meta128, xin, gate_up_proj, down_proj):
    npad, h = xin.shape
    ff = down_proj.shape[1]
    kf = ff // _FFB
    grid_spec = pltpu.PrefetchScalarGridSpec(
        num_scalar_prefetch=1,
        grid=(_NUM_EXPERTS, kf),
        in_specs=[
            pl.BlockSpec((npad, h), lambda e, f, m: (0, 0)),
            pl.BlockSpec((1, h, _FFB), lambda e, f, m: (e, 0, f)),
            pl.BlockSpec((1, h, _FFB), lambda e, f, m: (e, 0, f + kf)),
            pl.BlockSpec((1, _FFB, h), lambda e, f, m: (e, f, 0)),
        ],
        out_specs=pl.BlockSpec((npad, h), lambda e, f, m: (0, 0)),
    )
    return pl.pallas_call(
        _expert_body,
        grid_spec=grid_spec,
        out_shape=jax.ShapeDtypeStruct((npad, h), jnp.float32),
        compiler_params=pltpu.CompilerParams(
            vmem_limit_bytes=100 * 1024 * 1024,
        ),
    )(meta128, xin, gate_up_proj, gate_up_proj, down_proj)


def _shared_body(hid_ref, rout_ref, nrm_ref, wg_ref, wu_ref, wd_ref, o_ref):
    x = nrm_ref[...]
    g = jnp.dot(x, wg_ref[...], preferred_element_type=jnp.float32)
    u = jnp.dot(x, wu_ref[...], preferred_element_type=jnp.float32)
    s = jnp.dot(g * jax.nn.sigmoid(g) * u, wd_ref[...],
                preferred_element_type=jnp.float32)
    o_ref[...] = hid_ref[...] + rout_ref[...] + s


def _shared_ffn(hid, routed, nrm, wg, wu, wd):
    t, h = nrm.shape
    ff = wg.shape[1]
    nb = t // _BLK
    return pl.pallas_call(
        _shared_body,
        grid=(nb,),
        in_specs=[
            pl.BlockSpec((_BLK, h), lambda b: (b, 0)),
            pl.BlockSpec((_BLK, h), lambda b: (b, 0)),
            pl.BlockSpec((_BLK, h), lambda b: (b, 0)),
            pl.BlockSpec((h, ff), lambda b: (0, 0)),
            pl.BlockSpec((h, ff), lambda b: (0, 0)),
            pl.BlockSpec((ff, h), lambda b: (0, 0)),
        ],
        out_specs=pl.BlockSpec((_BLK, h), lambda b: (b, 0)),
        out_shape=jax.ShapeDtypeStruct((t, h), jnp.float32),
        compiler_params=pltpu.CompilerParams(
            vmem_limit_bytes=100 * 1024 * 1024,
        ),
    )(hid, routed, nrm, wg, wu, wd)


def kernel(hidden_states, ln_gamma, ln_beta, router_w, gate_up_proj,
           down_proj, shared_gate, shared_up, shared_down):
    b, s, h = hidden_states.shape
    t = b * s
    e = router_w.shape[1]
    nb = t // _BLK + e
    npad = nb * _BLK
    hs = hidden_states.reshape(t, h)

    nrm, scl, p3, meta3 = _ln_router(hs, ln_gamma, ln_beta, router_w)
    p = p3.reshape(t)
    meta128 = meta3.reshape(-1)

    xin_sorted = _make_sc_scatter(t, npad, h)(scl, p)
    routed_sorted = _grouped_ffn(meta128, xin_sorted, gate_up_proj, down_proj)
    routed = _make_sc_gather(t, h)(routed_sorted, p)

    out = _shared_ffn(hs, routed, nrm, shared_gate, shared_up, shared_down)
    return out.reshape(b, s, h)


# speedup vs baseline: 1.1298x; 1.0180x over previous
import functools

import jax
import jax.numpy as jnp
from jax import lax
from jax.experimental import pallas as pl
from jax.experimental.pallas import tpu as pltpu
from jax.experimental.pallas import tpu_sc as plsc

_BLK = 128
_NEG = -1e30
_NUM_EXPERTS = 8


_LNB = 512


def _ln_router_body(x_ref, g_ref, b_ref, wp_ref,
                    nrm_ref, scl_ref, p_ref, m_ref, obuf, cumb, runn):
    f32 = jnp.float32
    i = pl.program_id(0)
    nblk = obuf.shape[0]
    c_i = lax.broadcasted_iota(jnp.int32, (_LNB, 128), 1)
    ones_row = jnp.ones((1, _LNB), f32)

    @pl.when(i == 0)
    def _():
        runn[...] = jnp.zeros((1, 128), f32)

    @pl.when(i < nblk)
    def _():
        x = x_ref[...]
        mu = jnp.mean(x, axis=1, keepdims=True)
        xc = x - mu
        var = jnp.mean(xc * xc, axis=1, keepdims=True)
        nrm = xc * lax.rsqrt(var + 1e-5)
        nrm = nrm * g_ref[...] + b_ref[...]
        logits = jnp.dot(nrm, wp_ref[...], preferred_element_type=f32)
        logits = jnp.where(c_i < _NUM_EXPERTS, logits, _NEG)
        m = jnp.max(logits, axis=1, keepdims=True)
        eid_col = jnp.min(jnp.where(logits == m, c_i, 128), axis=1,
                          keepdims=True)
        nrm_ref[...] = nrm
        scl_ref[...] = nrm * jax.nn.sigmoid(m)
        onehot = (c_i == eid_col).astype(f32)
        cnt = jnp.dot(ones_row, onehot, preferred_element_type=f32)
        cumb[i] = runn[...]
        obuf[i] = onehot.astype(jnp.bfloat16)
        runn[...] = runn[...] + cnt

    @pl.when(i == nblk)
    def _():
        rt = lax.broadcasted_iota(jnp.int32, (_LNB, _LNB), 0)
        ct = lax.broadcasted_iota(jnp.int32, (_LNB, _LNB), 1)
        lstrict = (ct < rt).astype(f32)
        r_e = lax.broadcasted_iota(jnp.int32, (128, 128), 0)
        c_e = lax.broadcasted_iota(jnp.int32, (128, 128), 1)
        ustrict = (r_e < c_e).astype(f32)
        counts = runn[...]
        pc = jnp.floor((counts + (_BLK - 1.0)) / _BLK) * _BLK
        poff = jnp.dot(pc, ustrict, preferred_element_type=f32)

        shift_e = (r_e + _NUM_EXPERTS == c_e).astype(f32)
        lane = lax.broadcasted_iota(jnp.int32, (1, 128), 1)
        packed = jnp.where(lane < _NUM_EXPERTS, poff,
                           jnp.dot(pc / _BLK, shift_e, preferred_element_type=f32))
        m_ref[...] = packed.astype(jnp.int32).reshape(1, 1, 128)

        def pass1(b, carry):
            onehot = obuf[b].astype(f32)
            rank_before = jnp.dot(lstrict, onehot, preferred_element_type=f32)
            val = rank_before + poff + cumb[b]
            pcol = jnp.sum(onehot * val, axis=1)
            p_ref[b] = pcol.reshape(_LNB // 128, 128).astype(jnp.int32)
            return carry

        lax.fori_loop(0, nblk, pass1, 0)


def _ln_router(hs, gamma, beta, router_w):
    t, h = hs.shape
    nb = t // _LNB
    rows = _LNB // 128
    wp = jnp.zeros((h, 128), jnp.float32).at[:, : router_w.shape[1]].set(router_w)
    return pl.pallas_call(
        _ln_router_body,
        grid=(nb + 1,),
        in_specs=[
            pl.BlockSpec((_LNB, h), lambda b: (jnp.minimum(b, nb - 1), 0)),
            pl.BlockSpec((1, h), lambda b: (0, 0)),
            pl.BlockSpec((1, h), lambda b: (0, 0)),
            pl.BlockSpec((h, 128), lambda b: (0, 0)),
        ],
        out_specs=[
            pl.BlockSpec((_LNB, h), lambda b: (jnp.minimum(b, nb - 1), 0)),
            pl.BlockSpec((_LNB, h), lambda b: (jnp.minimum(b, nb - 1), 0)),
            pl.BlockSpec((nb, rows, 128), lambda b: (0, 0, 0)),
            pl.BlockSpec((1, 1, 128), lambda b: (0, 0, 0)),
        ],
        out_shape=[
            jax.ShapeDtypeStruct((t, h), jnp.float32),
            jax.ShapeDtypeStruct((t, h), jnp.float32),
            jax.ShapeDtypeStruct((nb, rows, 128), jnp.int32),
            jax.ShapeDtypeStruct((1, 1, 128), jnp.int32),
        ],
        scratch_shapes=[
            pltpu.VMEM((nb, _LNB, 128), jnp.bfloat16),
            pltpu.VMEM((nb, 1, 128), jnp.float32),
            pltpu.VMEM((1, 128), jnp.float32),
        ],
    )(hs, gamma.reshape(1, h), beta.reshape(1, h), wp)


def _make_sc_scatter(n_src, n_out, n_cols):
    info = plsc.get_sparse_core_info()
    nc, ns = info.num_cores, info.num_subcores
    nw = nc * ns
    bpw = n_src // nw
    mesh = plsc.VectorSubcoreMesh(core_axis_name="c", subcore_axis_name="s")

    @functools.partial(
        pl.kernel,
        mesh=mesh,
        out_type=jax.ShapeDtypeStruct((n_out, n_cols), jnp.float32),
        scratch_types=[
            pltpu.VMEM((bpw,), jnp.int32),
            pltpu.VMEM((bpw, n_cols), jnp.float32),
            pltpu.SemaphoreType.DMA,
        ],
    )
    def scatter_k(table_hbm, idx_hbm, out_hbm, idx_v, rows_v, sem):
        wid = lax.axis_index("s") * nc + lax.axis_index("c")
        base = wid * bpw
        pltpu.sync_copy(idx_hbm.at[pl.ds(base, bpw)], idx_v)
        pltpu.sync_copy(table_hbm.at[pl.ds(base, bpw)], rows_v)
        pltpu.async_copy(rows_v, out_hbm.at[idx_v], sem).wait()

    return scatter_k


def _make_sc_gather(n_out, n_cols):
    info = plsc.get_sparse_core_info()
    nc, ns = info.num_cores, info.num_subcores
    nw = nc * ns
    bpw = n_out // nw
    mesh = plsc.VectorSubcoreMesh(core_axis_name="c", subcore_axis_name="s")

    @functools.partial(
        pl.kernel,
        mesh=mesh,
        out_type=jax.ShapeDtypeStruct((n_out, n_cols), jnp.float32),
        scratch_types=[
            pltpu.VMEM((bpw,), jnp.int32),
            pltpu.VMEM((bpw, n_cols), jnp.float32),
            pltpu.SemaphoreType.DMA,
        ],
    )
    def gather_k(table_hbm, idx_hbm, out_hbm, idx_v, rows_v, sem):
        wid = lax.axis_index("s") * nc + lax.axis_index("c")
        base = wid * bpw
        pltpu.sync_copy(idx_hbm.at[pl.ds(base, bpw)], idx_v)
        pltpu.async_copy(table_hbm.at[idx_v], rows_v, sem).wait()
        pltpu.sync_copy(rows_v, out_hbm.at[pl.ds(base, bpw)])

    return gather_k


_FFB = 1024


def _expert_body(m_ref, x_ref, wg_ref, wu_ref, wd_ref, o_ref):
    e = pl.program_id(0)
    f = pl.program_id(1)
    poff = m_ref[e]
    nblk = m_ref[_NUM_EXPERTS + e]

    def body(j, carry):
        r0 = pl.multiple_of(poff + j * _BLK, _BLK)
        x = x_ref[pl.ds(r0, _BLK), :]
        g = jnp.dot(x, wg_ref[0], preferred_element_type=jnp.float32)
        u = jnp.dot(x, wu_ref[0], preferred_element_type=jnp.float32)
        part = jnp.dot(g * jax.nn.sigmoid(g) * u, wd_ref[0],
                       preferred_element_type=jnp.float32)

        @pl.when(f == 0)
        def _():
            o_ref[pl.ds(r0, _BLK), :] = part

        @pl.when(f != 0)
        def _():
            o_ref[pl.ds(r0, _BLK), :] = o_ref[pl.ds(r0, _BLK), :] + part

        return carry

    lax.fori_loop(0, nblk, body, 0)


def _grouped_ffn(meta128, xin, gate_up_proj, down_proj):
    npad, h = xin.shape
    ff = down_proj.shape[1]
    kf = ff // _FFB
    grid_spec = pltpu.PrefetchScalarGridSpec(
        num_scalar_prefetch=1,
        grid=(_NUM_EXPERTS, kf),
        in_specs=[
            pl.BlockSpec((npad, h), lambda e, f, m: (0, 0)),
            pl.BlockSpec((1, h, _FFB), lambda e, f, m: (e, 0, f)),
            pl.BlockSpec((1, h, _FFB), lambda e, f, m: (e, 0, f + kf)),
            pl.BlockSpec((1, _FFB, h), lambda e, f, m: (e, f, 0)),
        ],
        out_specs=pl.BlockSpec((npad, h), lambda e, f, m: (0, 0)),
    )
    return pl.pallas_call(
        _expert_body,
        grid_spec=grid_spec,
        out_shape=jax.ShapeDtypeStruct((npad, h), jnp.float32),
        compiler_params=pltpu.CompilerParams(
            vmem_limit_bytes=100 * 1024 * 1024,
        ),
    )(meta128, xin, gate_up_proj, gate_up_proj, down_proj)


def _shared_body(hid_ref, rout_ref, nrm_ref, wg_ref, wu_ref, wd_ref, o_ref):
    x = nrm_ref[...]
    g = jnp.dot(x, wg_ref[...], preferred_element_type=jnp.float32)
    u = jnp.dot(x, wu_ref[...], preferred_element_type=jnp.float32)
    s = jnp.dot(g * jax.nn.sigmoid(g) * u, wd_ref[...],
                preferred_element_type=jnp.float32)
    o_ref[...] = hid_ref[...] + rout_ref[...] + s


def _shared_ffn(hid, routed, nrm, wg, wu, wd):
    t, h = nrm.shape
    ff = wg.shape[1]
    nb = t // _BLK
    return pl.pallas_call(
        _shared_body,
        grid=(nb,),
        in_specs=[
            pl.BlockSpec((_BLK, h), lambda b: (b, 0)),
            pl.BlockSpec((_BLK, h), lambda b: (b, 0)),
            pl.BlockSpec((_BLK, h), lambda b: (b, 0)),
            pl.BlockSpec((h, ff), lambda b: (0, 0)),
            pl.BlockSpec((h, ff), lambda b: (0, 0)),
            pl.BlockSpec((ff, h), lambda b: (0, 0)),
        ],
        out_specs=pl.BlockSpec((_BLK, h), lambda b: (b, 0)),
        out_shape=jax.ShapeDtypeStruct((t, h), jnp.float32),
        compiler_params=pltpu.CompilerParams(
            vmem_limit_bytes=100 * 1024 * 1024,
        ),
    )(hid, routed, nrm, wg, wu, wd)


def kernel(hidden_states, ln_gamma, ln_beta, router_w, gate_up_proj,
           down_proj, shared_gate, shared_up, shared_down):
    b, s, h = hidden_states.shape
    t = b * s
    e = router_w.shape[1]
    nb = t // _BLK + e
    npad = nb * _BLK
    hs = hidden_states.reshape(t, h)

    nrm, scl, p3, meta3 = _ln_router(hs, ln_gamma, ln_beta, router_w)
    p = p3.reshape(t)
    meta128 = meta3.reshape(-1)

    xin_sorted = _make_sc_scatter(t, npad, h)(scl, p)
    routed_sorted = _grouped_ffn(meta128, xin_sorted, gate_up_proj, down_proj)
    routed = _make_sc_gather(t, h)(routed_sorted, p)

    out = _shared_ffn(hs, routed, nrm, shared_gate, shared_up, shared_down)
    return out.reshape(b, s, h)
